# Initial kernel scaffold; baseline (speedup 1.0000x reference)
import jax
import jax.numpy as jnp
from jax.experimental import pallas as pl


def kernel(x, edge_index, mask_nodes, mask_token, W_enc1, b_enc1, W_enc2, b_enc2, Wc0, bc0, Wc1, bc1, Wc2, bc2, Wp1, bp1, Wp2, bp2):
    raise NotImplementedError("write your pallas kernel here")


# speedup vs baseline: 7.1833x; 7.1833x over previous
import functools

import jax
import jax.numpy as jnp
from jax import lax
from jax.experimental import pallas as pl
from jax.experimental.pallas import tpu as pltpu
from jax.experimental.pallas import tpu_sc as plsc

N = 10000
E = 320000
DIN = 128
DH = 256
HALF = 128
NP = 10240
TRASH = NP - 1

NTILES = 16
CH = 128
NCH = 160
SUP = 16
NSB = NCH // SUP
EP = NTILES * NCH * CH
NCHA = 79
EPA = 2 * NTILES * NCHA * CH
NMASKP = 2048
ROWS_PT = NP // NTILES

R = 512
NB = NP // R


def _sc_prep(dstA, maskA):
    return _make_sc_prep()(dstA, maskA)


def _sc_prep_sim(dstA, maskA):
    deg = jnp.stack([
        jnp.zeros((NP,), jnp.float32).at[dstA[cc].reshape(-1)].add(1.0)
        for cc in range(2)
    ])
    deg = jnp.broadcast_to(deg[:, :, None], (2, NP, HALF))
    flag = jnp.zeros((NP,), jnp.float32).at[maskA.reshape(-1)].add(1.0)
    flag = jnp.broadcast_to(flag[:, None], (NP, HALF))
    return deg, flag


def _sc_spmm_sim(srcE, dstE, tab):
    src = srcE.reshape(-1)
    dst = dstE.reshape(-1)
    return jnp.stack([
        jnp.zeros((NP, HALF), jnp.float32).at[dst].add(tab[2 * src + cc])
        for cc in range(2)
    ])


def _sc_spmm(srcE, dstE, tab):
    return _make_sc_spmm()(srcE, dstE, tab)


@functools.cache
def _mesh():
    return plsc.VectorSubcoreMesh(core_axis_name="c", subcore_axis_name="s")


@functools.cache
def _make_sc_prep():
    return functools.partial(
        pl.kernel,
        out_type=(
            jax.ShapeDtypeStruct((2, NP, HALF), jnp.float32),
            jax.ShapeDtypeStruct((NP, HALF), jnp.float32),
        ),
        mesh=_mesh(),
        scratch_types=[
            pltpu.VMEM((NCHA, CH), jnp.int32),
            pltpu.VMEM((1, CH), jnp.int32),
            pltpu.VMEM((CH, HALF), jnp.float32),
            pltpu.VMEM_SHARED((NP, HALF), jnp.float32),
        ],
    )(_sc_prep_body)


def _sc_prep_body(dstA, maskA, deg_out, flag_out, dst_stage, mask_stage,
                  ones_buf, acc_sh):
    c = lax.axis_index("c")
    s = lax.axis_index("s")

    def _fill(i, val):
        for k in range(HALF // 16):
            ones_buf[i, pl.ds(k * 16, 16)] = jnp.full((16,), val, jnp.float32)
        return val

    def _zero_acc():
        lax.fori_loop(0, CH, _fill, 0.0)
        for t in range(ROWS_PT // CH):
            pltpu.sync_copy(ones_buf, acc_sh.at[pl.ds(s * ROWS_PT + t * CH, CH)])
        lax.fori_loop(0, CH, _fill, 1.0)

    _zero_acc()
    plsc.subcore_barrier()
    pltpu.sync_copy(dstA.at[c, s], dst_stage)

    def _step(j, carry):
        pltpu.sync_copy(ones_buf, acc_sh.at[dst_stage.at[j]], add=True)
        return carry

    lax.fori_loop(0, NCHA, _step, 0)
    plsc.subcore_barrier()
    for t in range(ROWS_PT // CH):
        off = s * ROWS_PT + t * CH
        pltpu.sync_copy(acc_sh.at[pl.ds(off, CH)], deg_out.at[c, pl.ds(off, CH)])

    _zero_acc()
    plsc.subcore_barrier()

    @pl.when(c == 0)
    def _():
        pltpu.sync_copy(maskA.at[s], mask_stage.at[0])
        pltpu.sync_copy(ones_buf, acc_sh.at[mask_stage.at[0]], add=True)

    plsc.subcore_barrier()

    @pl.when(c == 0)
    def _():
        for t in range(ROWS_PT // CH):
            off = s * ROWS_PT + t * CH
            pltpu.sync_copy(acc_sh.at[pl.ds(off, CH)], flag_out.at[pl.ds(off, CH)])


@functools.cache
def _make_sc_spmm():
    return functools.partial(
        pl.kernel,
        out_type=jax.ShapeDtypeStruct((2, NP, HALF), jnp.float32),
        mesh=_mesh(),
        scratch_types=[
            pltpu.VMEM((SUP, CH), jnp.int32),
            pltpu.VMEM((SUP, CH), jnp.int32),
            pltpu.VMEM((2, CH), jnp.int32),
            pltpu.VMEM((2, CH, HALF), jnp.float32),
            pltpu.VMEM_SHARED((NP, HALF), jnp.float32),
            pltpu.SemaphoreType.DMA,
            pltpu.SemaphoreType.DMA,
        ],
    )(_sc_spmm_body)


def _sc_spmm_body(srcE, dstE, tab, acc_out, src_st, dst_st, idx_buf,
                  rows_buf, acc_sh, sem0, sem1):
    c = lax.axis_index("c")
    s = lax.axis_index("s")
    sems = (sem0, sem1)

    def _zero(i, carry):
        for k in range(HALF // 16):
            rows_buf[0, i, pl.ds(k * 16, 16)] = jnp.zeros((16,), jnp.float32)
        return carry

    lax.fori_loop(0, CH, _zero, 0)
    for t in range(ROWS_PT // CH):
        pltpu.sync_copy(rows_buf.at[0], acc_sh.at[pl.ds(s * ROWS_PT + t * CH, CH)])
    plsc.subcore_barrier()

    def _fire(jj, b):
        for k in range(CH // 16):
            idx_buf[b, pl.ds(k * 16, 16)] = src_st[jj, pl.ds(k * 16, 16)] * 2 + c
        pltpu.async_copy(tab.at[idx_buf.at[b]], rows_buf.at[b], sems[b])

    def _block(sb, carry):
        pltpu.sync_copy(srcE.at[s, pl.ds(sb * SUP, SUP)], src_st)
        pltpu.sync_copy(dstE.at[s, pl.ds(sb * SUP, SUP)], dst_st)
        _fire(0, 0)

        def _inner(q, carry2):
            for b in range(2):
                jj = q * 2 + b
                pltpu.make_async_copy(tab.at[idx_buf.at[b]], rows_buf.at[b],
                                      sems[b]).wait()

                @pl.when(jj + 1 < SUP)
                def _():
                    _fire(jj + 1, 1 - b)

                pltpu.sync_copy(rows_buf.at[b], acc_sh.at[dst_st.at[jj]],
                                add=True)
            return carry2

        lax.fori_loop(0, SUP // 2, _inner, 0)
        return carry

    lax.fori_loop(0, NSB, _block, 0)

    plsc.subcore_barrier()
    for t in range(ROWS_PT // CH):
        off = s * ROWS_PT + t * CH
        pltpu.sync_copy(acc_sh.at[pl.ds(off, CH)], acc_out.at[c, pl.ds(off, CH)])


def _dinv_of(d0_ref, d1_ref):
    return lax.rsqrt(d0_ref[0, :, 0:1] + d1_ref[0, :, 0:1] + 1.0)


def _rows_of(i):
    return i * R + lax.broadcasted_iota(jnp.int32, (R, 1), 0)


def _enc_body(x_ref, fl_ref, d0_ref, d1_ref, mt_ref, w1_ref, b1_ref, w2_ref,
              b2_ref, wc0_ref, out_ref):
    i = pl.program_id(0)
    dinv = _dinv_of(d0_ref, d1_ref)
    fl = fl_ref[:, 0:1] > 0.5
    xm = jnp.where(fl, mt_ref[...], x_ref[...])
    h1 = jnp.maximum(jnp.dot(xm, w1_ref[...]) + b1_ref[...], 0.0)
    h = jnp.dot(h1, w2_ref[...]) + b2_ref[...]
    hw = jnp.dot(h, wc0_ref[...])
    out_ref[...] = jnp.where(_rows_of(i) < N, dinv * hw, 0.0)


def _dm_body(a0_ref, a1_ref, hwp_ref, d0_ref, d1_ref, b_ref, w_ref, out_ref):
    i = pl.program_id(0)
    dinv = _dinv_of(d0_ref, d1_ref)
    acc = jnp.concatenate([a0_ref[0], a1_ref[0]], axis=1)
    h1 = jnp.maximum(dinv * acc + dinv * hwp_ref[...] + b_ref[...], 0.0)
    hw = jnp.dot(h1, w_ref[...])
    out_ref[...] = jnp.where(_rows_of(i) < N, dinv * hw, 0.0)


def _fin_body(a0_ref, a1_ref, hwp_ref, d0_ref, d1_ref, b_ref, wp1_ref, bp1_ref,
              wp2_ref, bp2_ref, h3_ref, g_ref, ssum, smax):
    i = pl.program_id(0)
    dinv = _dinv_of(d0_ref, d1_ref)
    acc = jnp.concatenate([a0_ref[0], a1_ref[0]], axis=1)
    h3 = jnp.maximum(dinv * acc + dinv * hwp_ref[...] + b_ref[...], 0.0)
    valid = _rows_of(i) < N
    h3z = jnp.where(valid, h3, 0.0)
    h3_ref[...] = h3z
    psum = jnp.sum(h3z, axis=0, keepdims=True)
    pmax = jnp.max(jnp.where(valid, h3, -jnp.inf), axis=0, keepdims=True)

    @pl.when(i == 0)
    def _():
        ssum[0:1, :] = psum
        smax[0:1, :] = pmax

    @pl.when(i > 0)
    def _():
        ssum[0:1, :] = ssum[0:1, :] + psum
        smax[0:1, :] = jnp.maximum(smax[0:1, :], pmax)

    @pl.when(i == NB - 1)
    def _():
        g = jnp.concatenate([ssum[0:1, :] / float(N), smax[0:1, :]], axis=1)
        g1 = jnp.maximum(jnp.dot(g, wp1_ref[...]) + bp1_ref[...], 0.0)
        ge = jnp.dot(g1, wp2_ref[...]) + bp2_ref[...]
        g_ref[...] = jnp.broadcast_to(ge, (8, DIN))


def _full(shape):
    return pl.BlockSpec(shape, lambda i: tuple(0 for _ in shape))


def _enc_call(xp, flag, deg, mask_token, w1, b1, w2, b2, wc0):
    return pl.pallas_call(
        _enc_body,
        grid=(NB,),
        in_specs=[
            pl.BlockSpec((R, DIN), lambda i: (i, 0)),
            pl.BlockSpec((R, HALF), lambda i: (i, 0)),
            pl.BlockSpec((1, R, HALF), lambda i: (0, i, 0)),
            pl.BlockSpec((1, R, HALF), lambda i: (1, i, 0)),
            _full((1, DIN)),
            _full((DIN, DH)),
            _full((1, DH)),
            _full((DH, DH)),
            _full((1, DH)),
            _full((DH, DH)),
        ],
        out_specs=pl.BlockSpec((R, DH), lambda i: (i, 0)),
        out_shape=jax.ShapeDtypeStruct((NP, DH), jnp.float32),
    )(xp, flag, deg, deg, mask_token, w1, b1.reshape(1, -1), w2,
      b2.reshape(1, -1), wc0)


def _dm_call(acc, hwp, deg, b, w_next):
    return pl.pallas_call(
        _dm_body,
        grid=(NB,),
        in_specs=[
            pl.BlockSpec((1, R, HALF), lambda i: (0, i, 0)),
            pl.BlockSpec((1, R, HALF), lambda i: (1, i, 0)),
            pl.BlockSpec((R, DH), lambda i: (i, 0)),
            pl.BlockSpec((1, R, HALF), lambda i: (0, i, 0)),
            pl.BlockSpec((1, R, HALF), lambda i: (1, i, 0)),
            _full((1, DH)),
            _full((DH, DH)),
        ],
        out_specs=pl.BlockSpec((R, DH), lambda i: (i, 0)),
        out_shape=jax.ShapeDtypeStruct((NP, DH), jnp.float32),
    )(acc, acc, hwp, deg, deg, b.reshape(1, -1), w_next)


def _fin_call(acc, hwp, deg, b, wp1, bp1, wp2, bp2):
    return pl.pallas_call(
        _fin_body,
        grid=(NB,),
        in_specs=[
            pl.BlockSpec((1, R, HALF), lambda i: (0, i, 0)),
            pl.BlockSpec((1, R, HALF), lambda i: (1, i, 0)),
            pl.BlockSpec((R, DH), lambda i: (i, 0)),
            pl.BlockSpec((1, R, HALF), lambda i: (0, i, 0)),
            pl.BlockSpec((1, R, HALF), lambda i: (1, i, 0)),
            _full((1, DH)),
            _full((2 * DH, DH)),
            _full((1, DH)),
            _full((DH, DIN)),
            _full((1, DIN)),
        ],
        out_specs=[
            pl.BlockSpec((R, DH), lambda i: (i, 0)),
            pl.BlockSpec((8, DIN), lambda i: (0, 0)),
        ],
        out_shape=[
            jax.ShapeDtypeStruct((NP, DH), jnp.float32),
            jax.ShapeDtypeStruct((8, DIN), jnp.float32),
        ],
        scratch_shapes=[
            pltpu.VMEM((8, DH), jnp.float32),
            pltpu.VMEM((8, DH), jnp.float32),
        ],
    )(acc, acc, hwp, deg, deg, b.reshape(1, -1), wp1, bp1.reshape(1, -1), wp2,
      bp2.reshape(1, -1))


def kernel(x, edge_index, mask_nodes, mask_token, W_enc1, b_enc1, W_enc2,
           b_enc2, Wc0, bc0, Wc1, bc1, Wc2, bc2, Wp1, bp1, Wp2, bp2):
    src = edge_index[0].astype(jnp.int32)
    dst = edge_index[1].astype(jnp.int32)
    srcE = jnp.concatenate(
        [src, jnp.full((EP - E,), TRASH, jnp.int32)]).reshape(NTILES, NCH, CH)
    dstE = jnp.concatenate(
        [dst, jnp.full((EP - E,), TRASH, jnp.int32)]).reshape(NTILES, NCH, CH)
    dstA = jnp.concatenate(
        [dst, jnp.full((EPA - E,), TRASH, jnp.int32)]
    ).reshape(2, NTILES, NCHA, CH)
    maskA = jnp.concatenate(
        [mask_nodes.astype(jnp.int32),
         jnp.full((NMASKP - mask_nodes.shape[0],), TRASH, jnp.int32)]
    ).reshape(NTILES, CH)
    xp = jnp.concatenate([x, jnp.zeros((NP - N, DIN), x.dtype)], axis=0)

    deg, flag = _sc_prep(dstA, maskA)
    hwp0 = _enc_call(xp, flag, deg, mask_token, W_enc1, b_enc1, W_enc2, b_enc2,
                     Wc0)
    acc0 = _sc_spmm(srcE, dstE, hwp0.reshape(2 * NP, HALF))
    hwp1 = _dm_call(acc0, hwp0, deg, bc0, Wc1)
    acc1 = _sc_spmm(srcE, dstE, hwp1.reshape(2 * NP, HALF))
    hwp2 = _dm_call(acc1, hwp1, deg, bc1, Wc2)
    acc2 = _sc_spmm(srcE, dstE, hwp2.reshape(2 * NP, HALF))
    h3, gemb = _fin_call(acc2, hwp2, deg, bc2, Wp1, bp1, Wp2, bp2)
    return (gemb[0:1, :], h3[:N, :])
